# baseline (speedup 1.0000x reference)
import functools

import jax
import jax.numpy as jnp
from jax import lax
from jax.experimental import pallas as pl
from jax.experimental.pallas import tpu as pltpu
from jax.experimental.pallas import tpu_sc as plsc

NC = 2
NS = 16
NW = NC * NS
CH = 128
D = 128
R = 1024


def _rup(a, b):
    return -(-a // b) * b


def _mesh():
    return plsc.VectorSubcoreMesh(
        core_axis_name="c", subcore_axis_name="s", num_cores=NC, num_subcores=NS
    )


def _deg_kernel(np_, tch):
    share = np_ // NS

    @functools.partial(
        pl.kernel,
        out_type=jax.ShapeDtypeStruct((NC, np_), jnp.float32),
        mesh=_mesh(),
        compiler_params=pltpu.CompilerParams(needs_layout_passes=False),
        scratch_types=[
            pltpu.VMEM((tch, CH), jnp.int32),
            pltpu.VMEM((CH,), jnp.float32),
            pltpu.VMEM((share,), jnp.float32),
            pltpu.VMEM_SHARED((np_,), jnp.float32),
        ],
    )
    def k(dst3, degp, idx_v, ones_v, obuf_v, acc):
        cid = lax.axis_index("c")
        sid = lax.axis_index("s")
        w = cid * NS + sid

        def fill_ones(i, _):
            ones_v[pl.ds(i * 16, 16)] = jnp.ones((16,), jnp.float32)
            return 0

        lax.fori_loop(0, CH // 16, fill_ones, 0)

        def fill_zero(i, _):
            obuf_v[pl.ds(i * 16, 16)] = jnp.zeros((16,), jnp.float32)
            return 0

        lax.fori_loop(0, share // 16, fill_zero, 0)
        pltpu.sync_copy(obuf_v, acc.at[pl.ds(sid * share, share)])
        pltpu.sync_copy(dst3.at[w], idx_v)
        plsc.subcore_barrier()

        def chunk(j, _):
            pltpu.sync_copy(ones_v, acc.at[idx_v.at[j]], add=True)
            return 0

        lax.fori_loop(0, tch, chunk, 0)
        plsc.subcore_barrier()
        pltpu.sync_copy(acc.at[pl.ds(sid * share, share)], obuf_v)
        pltpu.sync_copy(obuf_v, degp.at[cid].at[pl.ds(sid * share, share)])

    return k


def _agg_kernel(np_, tch, nbuf=2, nhalf=2):
    share = np_ // NS
    hlf = tch // nhalf

    @functools.partial(
        pl.kernel,
        out_type=jax.ShapeDtypeStruct((NC, np_, D), jnp.float32),
        mesh=_mesh(),
        compiler_params=pltpu.CompilerParams(needs_layout_passes=False),
        scratch_types=[
            pltpu.VMEM((hlf, CH), jnp.int32),
            pltpu.VMEM((hlf, CH), jnp.int32),
            pltpu.VMEM((nbuf, CH, D), jnp.float32),
            pltpu.VMEM_SHARED((np_, D), jnp.float32),
            pltpu.SemaphoreType.DMA((nbuf,)),
        ],
    )
    def k(h_hbm, z_hbm, src3, dst3, out_hbm, srcv, dstv, rowb, acc, gsem):
        cid = lax.axis_index("c")
        sid = lax.axis_index("s")
        w = cid * NS + sid
        pltpu.sync_copy(z_hbm, acc.at[pl.ds(sid * share, share)])
        plsc.subcore_barrier()

        def run_half(h0):
            pltpu.sync_copy(src3.at[w].at[pl.ds(h0, hlf)], srcv)
            pltpu.sync_copy(dst3.at[w].at[pl.ds(h0, hlf)], dstv)
            for b in range(nbuf):
                pltpu.async_copy(h_hbm.at[srcv.at[b]], rowb.at[b], gsem.at[b])

            def grp(g, _):
                for b in range(nbuf):
                    j = g * nbuf + b
                    pltpu.make_async_copy(
                        h_hbm.at[srcv.at[j]], rowb.at[b], gsem.at[b]
                    ).wait()
                    pltpu.sync_copy(rowb.at[b], acc.at[dstv.at[j]], add=True)
                    nxt = j + nbuf

                    @pl.when(nxt < hlf)
                    def _():
                        pltpu.async_copy(
                            h_hbm.at[srcv.at[nxt]], rowb.at[b], gsem.at[b]
                        )

                return 0

            lax.fori_loop(0, hlf // nbuf, grp, 0)

        for h in range(nhalf):
            run_half(h * hlf)
        plsc.subcore_barrier()
        pltpu.sync_copy(
            acc.at[pl.ds(sid * share, share)],
            out_hbm.at[cid].at[pl.ds(sid * share, share)],
        )

    return k


def _scalar_agg_kernel(np_, tch):
    share = np_ // NS

    @functools.partial(
        pl.kernel,
        out_type=jax.ShapeDtypeStruct((NC, np_), jnp.float32),
        mesh=_mesh(),
        compiler_params=pltpu.CompilerParams(needs_layout_passes=False),
        scratch_types=[
            pltpu.VMEM((tch, CH), jnp.int32),
            pltpu.VMEM((tch, CH), jnp.int32),
            pltpu.VMEM((np_,), jnp.float32),
            pltpu.VMEM((CH,), jnp.float32),
            pltpu.VMEM((share,), jnp.float32),
            pltpu.VMEM_SHARED((np_,), jnp.float32),
        ],
    )
    def k(st_hbm, src3, dst3, out_hbm, srcv, dstv, table_v, chunk_v, obuf_v, acc):
        cid = lax.axis_index("c")
        sid = lax.axis_index("s")
        w = cid * NS + sid

        def fill_zero(i, _):
            obuf_v[pl.ds(i * 16, 16)] = jnp.zeros((16,), jnp.float32)
            return 0

        lax.fori_loop(0, share // 16, fill_zero, 0)
        pltpu.sync_copy(obuf_v, acc.at[pl.ds(sid * share, share)])
        pltpu.sync_copy(st_hbm, table_v)
        pltpu.sync_copy(src3.at[w], srcv)
        pltpu.sync_copy(dst3.at[w], dstv)
        plsc.subcore_barrier()

        def chunk(j, _):
            for kk in range(CH // 16):
                idx16 = srcv[j, pl.ds(kk * 16, 16)]
                chunk_v[pl.ds(kk * 16, 16)] = plsc.load_gather(table_v, [idx16])
            pltpu.sync_copy(chunk_v, acc.at[dstv.at[j]], add=True)
            return 0

        lax.fori_loop(0, tch, chunk, 0)
        plsc.subcore_barrier()
        pltpu.sync_copy(acc.at[pl.ds(sid * share, share)], obuf_v)
        pltpu.sync_copy(obuf_v, out_hbm.at[cid].at[pl.ds(sid * share, share)])

    return k


BK = 4096


def _edge_prep(n, e, ep, np_):
    def body(ei_ref, src_ref, dst_ref):
        i = pl.program_id(0)
        col = jax.lax.broadcasted_iota(jnp.int32, (1, BK), 1) + i * BK
        sv = ei_ref[0:1, :]
        dv = ei_ref[1:2, :]
        p = col - e
        src = jnp.where(col < e, sv, (p * 37) % n)
        dst = jnp.where(col < e, dv, n + p % (np_ - n))
        src_ref[...] = src.reshape(BK)
        dst_ref[...] = dst.reshape(BK)

    last = (e - 1) // BK
    return pl.pallas_call(
        body,
        grid=(ep // BK,),
        in_specs=[pl.BlockSpec((2, BK), lambda i: (0, jnp.minimum(i, last)))],
        out_specs=[
            pl.BlockSpec((BK,), lambda i: (i,)),
            pl.BlockSpec((BK,), lambda i: (i,)),
        ],
        out_shape=[
            jax.ShapeDtypeStruct((ep,), jnp.int32),
            jax.ShapeDtypeStruct((ep,), jnp.int32),
        ],
    )


def _scale_mm_body(x_ref, w_ref, degp_ref, h_ref, dis_ref):
    deg = 1.0 + degp_ref[0] + degp_ref[1]
    dis = lax.rsqrt(deg)
    h = jnp.dot(x_ref[...], w_ref[...], preferred_element_type=jnp.float32,
                precision=lax.Precision.HIGHEST)
    h_ref[...] = h * dis.reshape(R, 1)
    dis_ref[...] = dis


def _scale_mm(np_):
    return pl.pallas_call(
        _scale_mm_body,
        grid=(np_ // R,),
        in_specs=[
            pl.BlockSpec((R, D), lambda i: (i, 0)),
            pl.BlockSpec((D, D), lambda i: (0, 0)),
            pl.BlockSpec((NC, R), lambda i: (0, i)),
        ],
        out_specs=[
            pl.BlockSpec((R, D), lambda i: (i, 0)),
            pl.BlockSpec((R,), lambda i: (i,)),
        ],
        out_shape=[
            jax.ShapeDtypeStruct((np_, D), jnp.float32),
            jax.ShapeDtypeStruct((np_,), jnp.float32),
        ],
    )


def _mid_body(p_ref, h_ref, dis_ref, b1_ref, w2_ref, st_ref):
    a = p_ref[0] + p_ref[1] + h_ref[...]
    dis = dis_ref[...]
    h1 = dis.reshape(R, 1) * a + b1_ref[...]
    r = jnp.maximum(h1, 0.0)
    s = jnp.sum(r * w2_ref[...], axis=1)
    st_ref[...] = dis * s


def _mid(np_):
    return pl.pallas_call(
        _mid_body,
        grid=(np_ // R,),
        in_specs=[
            pl.BlockSpec((NC, R, D), lambda i: (0, i, 0)),
            pl.BlockSpec((R, D), lambda i: (i, 0)),
            pl.BlockSpec((R,), lambda i: (i,)),
            pl.BlockSpec((1, D), lambda i: (0, 0)),
            pl.BlockSpec((1, D), lambda i: (0, 0)),
        ],
        out_specs=pl.BlockSpec((R,), lambda i: (i,)),
        out_shape=jax.ShapeDtypeStruct((np_,), jnp.float32),
    )


def _final_body(p2_ref, st_ref, dis_ref, b2_ref, o_ref):
    t = p2_ref[0] + p2_ref[1] + st_ref[...]
    o = jnp.tanh(dis_ref[...] * t + b2_ref[0])
    o_ref[...] = o.reshape(R, 1)


def _final(np_, n):
    return pl.pallas_call(
        _final_body,
        grid=(np_ // R,),
        in_specs=[
            pl.BlockSpec((NC, R), lambda i: (0, i)),
            pl.BlockSpec((R,), lambda i: (i,)),
            pl.BlockSpec((R,), lambda i: (i,)),
            pl.BlockSpec((1,), lambda i: (0,)),
        ],
        out_specs=pl.BlockSpec((R, 1), lambda i: (i, 0)),
        out_shape=jax.ShapeDtypeStruct((n, 1), jnp.float32),
    )


def kernel(x, edge_index, W1, b1, W2, b2):
    n = x.shape[0]
    e = edge_index.shape[1]
    np_ = _rup(n + 128, NS * 128)
    tch = _rup(-(-e // (NW * CH)), 4)
    ep = NW * CH * tch
    npad = ep - e

    del npad
    src_f, dst_f = _edge_prep(n, e, ep, np_)(edge_index.astype(jnp.int32))
    src_p = src_f.reshape(NW, tch, CH)
    dst_p = dst_f.reshape(NW, tch, CH)

    degp = _deg_kernel(np_, tch)(dst_p)
    h_t, dis = _scale_mm(np_)(x, W1, degp)
    zrows = jnp.zeros((np_ // NS, D), jnp.float32)
    aggp = _agg_kernel(np_, tch)(h_t, zrows, src_p, dst_p)
    st = _mid(np_)(aggp, h_t, dis, b1.reshape(1, D), W2.reshape(1, D))
    agg2 = _scalar_agg_kernel(np_, tch)(st, src_p, dst_p)
    return _final(np_, n)(agg2, st, dis, b2)


# speedup vs baseline: 1.1354x; 1.1354x over previous
import functools

import jax
import jax.numpy as jnp
from jax import lax
from jax.experimental import pallas as pl
from jax.experimental.pallas import tpu as pltpu
from jax.experimental.pallas import tpu_sc as plsc

NC = 2
NS = 16
NW = NC * NS
CH = 128
D = 128
R = 1024


def _rup(a, b):
    return -(-a // b) * b


def _mesh():
    return plsc.VectorSubcoreMesh(
        core_axis_name="c", subcore_axis_name="s", num_cores=NC, num_subcores=NS
    )


def _deg_kernel(np_, tch):
    share = np_ // NS

    @functools.partial(
        pl.kernel,
        out_type=jax.ShapeDtypeStruct((NC, np_), jnp.float32),
        mesh=_mesh(),
        compiler_params=pltpu.CompilerParams(needs_layout_passes=False),
        scratch_types=[
            pltpu.VMEM((tch, CH), jnp.int32),
            pltpu.VMEM((CH,), jnp.float32),
            pltpu.VMEM((share,), jnp.float32),
            pltpu.VMEM_SHARED((np_,), jnp.float32),
        ],
    )
    def k(dst3, degp, idx_v, ones_v, obuf_v, acc):
        cid = lax.axis_index("c")
        sid = lax.axis_index("s")
        w = cid * NS + sid

        def fill_ones(i, _):
            ones_v[pl.ds(i * 16, 16)] = jnp.ones((16,), jnp.float32)
            return 0

        lax.fori_loop(0, CH // 16, fill_ones, 0)

        def fill_zero(i, _):
            obuf_v[pl.ds(i * 16, 16)] = jnp.zeros((16,), jnp.float32)
            return 0

        lax.fori_loop(0, share // 16, fill_zero, 0)
        pltpu.sync_copy(obuf_v, acc.at[pl.ds(sid * share, share)])
        pltpu.sync_copy(dst3.at[w], idx_v)
        plsc.subcore_barrier()

        def chunk(j, _):
            pltpu.sync_copy(ones_v, acc.at[idx_v.at[j]], add=True)
            return 0

        lax.fori_loop(0, tch, chunk, 0)
        plsc.subcore_barrier()
        pltpu.sync_copy(acc.at[pl.ds(sid * share, share)], obuf_v)
        pltpu.sync_copy(obuf_v, degp.at[cid].at[pl.ds(sid * share, share)])

    return k


def _agg_kernel(np_, tch, nbuf=2, nhalf=2):
    share = np_ // NS
    hlf = tch // nhalf

    @functools.partial(
        pl.kernel,
        out_type=jax.ShapeDtypeStruct((NC, np_, D), jnp.float32),
        mesh=_mesh(),
        compiler_params=pltpu.CompilerParams(needs_layout_passes=False),
        scratch_types=[
            pltpu.VMEM((hlf, CH), jnp.int32),
            pltpu.VMEM((hlf, CH), jnp.int32),
            pltpu.VMEM((nbuf, CH, D), jnp.float32),
            pltpu.VMEM_SHARED((np_, D), jnp.float32),
            pltpu.SemaphoreType.DMA((nbuf,)),
        ],
    )
    def k(h_hbm, z_hbm, src3, dst3, out_hbm, srcv, dstv, rowb, acc, gsem):
        cid = lax.axis_index("c")
        sid = lax.axis_index("s")
        w = cid * NS + sid
        pltpu.sync_copy(z_hbm, acc.at[pl.ds(sid * share, share)])
        plsc.subcore_barrier()

        def run_half(h0):
            pltpu.sync_copy(src3.at[w].at[pl.ds(h0, hlf)], srcv)
            pltpu.sync_copy(dst3.at[w].at[pl.ds(h0, hlf)], dstv)
            for b in range(nbuf):
                pltpu.async_copy(h_hbm.at[srcv.at[b]], rowb.at[b], gsem.at[b])

            def grp(g, _):
                for b in range(nbuf):
                    j = g * nbuf + b
                    pltpu.make_async_copy(
                        h_hbm.at[srcv.at[j]], rowb.at[b], gsem.at[b]
                    ).wait()
                    pltpu.sync_copy(rowb.at[b], acc.at[dstv.at[j]], add=True)
                    nxt = j + nbuf

                    @pl.when(nxt < hlf)
                    def _():
                        pltpu.async_copy(
                            h_hbm.at[srcv.at[nxt]], rowb.at[b], gsem.at[b]
                        )

                return 0

            lax.fori_loop(0, hlf // nbuf, grp, 0)

        for h in range(nhalf):
            run_half(h * hlf)
        plsc.subcore_barrier()
        pltpu.sync_copy(
            acc.at[pl.ds(sid * share, share)],
            out_hbm.at[cid].at[pl.ds(sid * share, share)],
        )

    return k


def _scalar_agg_kernel(np_, tch):
    share = np_ // NS

    @functools.partial(
        pl.kernel,
        out_type=jax.ShapeDtypeStruct((NC, np_), jnp.float32),
        mesh=_mesh(),
        compiler_params=pltpu.CompilerParams(needs_layout_passes=False),
        scratch_types=[
            pltpu.VMEM((tch, CH), jnp.int32),
            pltpu.VMEM((tch, CH), jnp.int32),
            pltpu.VMEM((np_,), jnp.float32),
            pltpu.VMEM((CH,), jnp.float32),
            pltpu.VMEM((share,), jnp.float32),
            pltpu.VMEM_SHARED((np_,), jnp.float32),
        ],
    )
    def k(st_hbm, src3, dst3, out_hbm, srcv, dstv, table_v, chunk_v, obuf_v, acc):
        cid = lax.axis_index("c")
        sid = lax.axis_index("s")
        w = cid * NS + sid

        def fill_zero(i, _):
            obuf_v[pl.ds(i * 16, 16)] = jnp.zeros((16,), jnp.float32)
            return 0

        lax.fori_loop(0, share // 16, fill_zero, 0)
        pltpu.sync_copy(obuf_v, acc.at[pl.ds(sid * share, share)])
        pltpu.sync_copy(st_hbm, table_v)
        pltpu.sync_copy(src3.at[w], srcv)
        pltpu.sync_copy(dst3.at[w], dstv)
        plsc.subcore_barrier()

        def chunk(j, _):
            for kk in range(CH // 16):
                idx16 = srcv[j, pl.ds(kk * 16, 16)]
                chunk_v[pl.ds(kk * 16, 16)] = plsc.load_gather(table_v, [idx16])
            pltpu.sync_copy(chunk_v, acc.at[dstv.at[j]], add=True)
            return 0

        lax.fori_loop(0, tch, chunk, 0)
        plsc.subcore_barrier()
        pltpu.sync_copy(acc.at[pl.ds(sid * share, share)], obuf_v)
        pltpu.sync_copy(obuf_v, out_hbm.at[cid].at[pl.ds(sid * share, share)])

    return k


BK = 32768


def _edge_prep(n, e, ep, np_):
    def body(ei_ref, src_ref, dst_ref):
        i = pl.program_id(0)
        col = jax.lax.broadcasted_iota(jnp.int32, (1, BK), 1) + i * BK
        sv = ei_ref[0:1, :]
        dv = ei_ref[1:2, :]
        p = col - e
        src = jnp.where(col < e, sv, (p * 37) % n)
        dst = jnp.where(col < e, dv, n + p % (np_ - n))
        src_ref[...] = src.reshape(BK)
        dst_ref[...] = dst.reshape(BK)

    last = (e - 1) // BK
    return pl.pallas_call(
        body,
        grid=(ep // BK,),
        in_specs=[pl.BlockSpec((2, BK), lambda i: (0, jnp.minimum(i, last)))],
        out_specs=[
            pl.BlockSpec((BK,), lambda i: (i,)),
            pl.BlockSpec((BK,), lambda i: (i,)),
        ],
        out_shape=[
            jax.ShapeDtypeStruct((ep,), jnp.int32),
            jax.ShapeDtypeStruct((ep,), jnp.int32),
        ],
    )


def _scale_mm_body(x_ref, w_ref, degp_ref, h_ref, dis_ref):
    deg = 1.0 + degp_ref[0] + degp_ref[1]
    dis = lax.rsqrt(deg)
    h = jnp.dot(x_ref[...], w_ref[...], preferred_element_type=jnp.float32,
                precision=lax.Precision.HIGHEST)
    h_ref[...] = h * dis.reshape(R, 1)
    dis_ref[...] = dis


def _scale_mm(np_):
    return pl.pallas_call(
        _scale_mm_body,
        grid=(np_ // R,),
        in_specs=[
            pl.BlockSpec((R, D), lambda i: (i, 0)),
            pl.BlockSpec((D, D), lambda i: (0, 0)),
            pl.BlockSpec((NC, R), lambda i: (0, i)),
        ],
        out_specs=[
            pl.BlockSpec((R, D), lambda i: (i, 0)),
            pl.BlockSpec((R,), lambda i: (i,)),
        ],
        out_shape=[
            jax.ShapeDtypeStruct((np_, D), jnp.float32),
            jax.ShapeDtypeStruct((np_,), jnp.float32),
        ],
    )


def _mid_body(p_ref, h_ref, dis_ref, b1_ref, w2_ref, st_ref):
    a = p_ref[0] + p_ref[1] + h_ref[...]
    dis = dis_ref[...]
    h1 = dis.reshape(R, 1) * a + b1_ref[...]
    r = jnp.maximum(h1, 0.0)
    s = jnp.sum(r * w2_ref[...], axis=1)
    st_ref[...] = dis * s


def _mid(np_):
    return pl.pallas_call(
        _mid_body,
        grid=(np_ // R,),
        in_specs=[
            pl.BlockSpec((NC, R, D), lambda i: (0, i, 0)),
            pl.BlockSpec((R, D), lambda i: (i, 0)),
            pl.BlockSpec((R,), lambda i: (i,)),
            pl.BlockSpec((1, D), lambda i: (0, 0)),
            pl.BlockSpec((1, D), lambda i: (0, 0)),
        ],
        out_specs=pl.BlockSpec((R,), lambda i: (i,)),
        out_shape=jax.ShapeDtypeStruct((np_,), jnp.float32),
    )


def _final_body(p2_ref, st_ref, dis_ref, b2_ref, o_ref):
    t = p2_ref[0] + p2_ref[1] + st_ref[...]
    o = jnp.tanh(dis_ref[...] * t + b2_ref[0])
    o_ref[...] = o.reshape(R, 1)


def _final(np_, n):
    return pl.pallas_call(
        _final_body,
        grid=(np_ // R,),
        in_specs=[
            pl.BlockSpec((NC, R), lambda i: (0, i)),
            pl.BlockSpec((R,), lambda i: (i,)),
            pl.BlockSpec((R,), lambda i: (i,)),
            pl.BlockSpec((1,), lambda i: (0,)),
        ],
        out_specs=pl.BlockSpec((R, 1), lambda i: (i, 0)),
        out_shape=jax.ShapeDtypeStruct((n, 1), jnp.float32),
    )


def kernel(x, edge_index, W1, b1, W2, b2):
    n = x.shape[0]
    e = edge_index.shape[1]
    np_ = _rup(n + 128, NS * 128)
    tch = _rup(-(-e // (NW * CH)), 4)
    ep = NW * CH * tch
    npad = ep - e

    del npad
    src_f, dst_f = _edge_prep(n, e, ep, np_)(edge_index.astype(jnp.int32))
    src_p = src_f.reshape(NW, tch, CH)
    dst_p = dst_f.reshape(NW, tch, CH)

    degp = _deg_kernel(np_, tch)(dst_p)
    h_t, dis = _scale_mm(np_)(x, W1, degp)
    zrows = jnp.zeros((np_ // NS, D), jnp.float32)
    aggp = _agg_kernel(np_, tch)(h_t, zrows, src_p, dst_p)
    st = _mid(np_)(aggp, h_t, dis, b1.reshape(1, D), W2.reshape(1, D))
    agg2 = _scalar_agg_kernel(np_, tch)(st, src_p, dst_p)
    return _final(np_, n)(agg2, st, dis, b2)


# speedup vs baseline: 1.1716x; 1.0318x over previous
import functools

import jax
import jax.numpy as jnp
from jax import lax
from jax.experimental import pallas as pl
from jax.experimental.pallas import tpu as pltpu
from jax.experimental.pallas import tpu_sc as plsc

NC = 2
NS = 16
NW = NC * NS
CH = 128
D = 128
R = 1024


def _rup(a, b):
    return -(-a // b) * b


def _mesh():
    return plsc.VectorSubcoreMesh(
        core_axis_name="c", subcore_axis_name="s", num_cores=NC, num_subcores=NS
    )


def _deg_kernel(np_, tch):
    share = np_ // NS

    @functools.partial(
        pl.kernel,
        out_type=jax.ShapeDtypeStruct((NC, np_), jnp.float32),
        mesh=_mesh(),
        compiler_params=pltpu.CompilerParams(needs_layout_passes=False),
        scratch_types=[
            pltpu.VMEM((tch, CH), jnp.int32),
            pltpu.VMEM((CH,), jnp.float32),
            pltpu.VMEM((share,), jnp.float32),
            pltpu.VMEM_SHARED((np_,), jnp.float32),
        ],
    )
    def k(dst3, degp, idx_v, ones_v, obuf_v, acc):
        cid = lax.axis_index("c")
        sid = lax.axis_index("s")
        w = cid * NS + sid

        def fill_ones(i, _):
            ones_v[pl.ds(i * 16, 16)] = jnp.ones((16,), jnp.float32)
            return 0

        lax.fori_loop(0, CH // 16, fill_ones, 0)

        def fill_zero(i, _):
            obuf_v[pl.ds(i * 16, 16)] = jnp.zeros((16,), jnp.float32)
            return 0

        lax.fori_loop(0, share // 16, fill_zero, 0)
        pltpu.sync_copy(obuf_v, acc.at[pl.ds(sid * share, share)])
        pltpu.sync_copy(dst3.at[w], idx_v)
        plsc.subcore_barrier()

        def chunk(j, _):
            pltpu.sync_copy(ones_v, acc.at[idx_v.at[j]], add=True)
            return 0

        lax.fori_loop(0, tch, chunk, 0)
        plsc.subcore_barrier()
        pltpu.sync_copy(acc.at[pl.ds(sid * share, share)], obuf_v)
        pltpu.sync_copy(obuf_v, degp.at[cid].at[pl.ds(sid * share, share)])

    return k


def _agg_kernel(np_, tch, nbuf=2, nhalf=2):
    share = np_ // NS
    hlf = tch // nhalf

    @functools.partial(
        pl.kernel,
        out_type=jax.ShapeDtypeStruct((NC, np_, D), jnp.float32),
        mesh=_mesh(),
        compiler_params=pltpu.CompilerParams(needs_layout_passes=False),
        scratch_types=[
            pltpu.VMEM((hlf, CH), jnp.int32),
            pltpu.VMEM((hlf, CH), jnp.int32),
            pltpu.VMEM((nbuf, CH, D), jnp.float32),
            pltpu.VMEM_SHARED((np_, D), jnp.float32),
            pltpu.SemaphoreType.DMA((nbuf,)),
        ],
    )
    def k(h_hbm, z_hbm, src3, dst3, out_hbm, srcv, dstv, rowb, acc, gsem):
        cid = lax.axis_index("c")
        sid = lax.axis_index("s")
        w = cid * NS + sid
        pltpu.sync_copy(z_hbm, acc.at[pl.ds(sid * share, share)])
        plsc.subcore_barrier()

        def run_half(h0):
            pltpu.sync_copy(src3.at[w].at[pl.ds(h0, hlf)], srcv)
            pltpu.sync_copy(dst3.at[w].at[pl.ds(h0, hlf)], dstv)
            for b in range(nbuf):
                pltpu.async_copy(h_hbm.at[srcv.at[b]], rowb.at[b], gsem.at[b])

            def grp(g, _):
                for b in range(nbuf):
                    j = g * nbuf + b
                    pltpu.make_async_copy(
                        h_hbm.at[srcv.at[j]], rowb.at[b], gsem.at[b]
                    ).wait()
                    pltpu.sync_copy(rowb.at[b], acc.at[dstv.at[j]], add=True)
                    nxt = j + nbuf

                    @pl.when(nxt < hlf)
                    def _():
                        pltpu.async_copy(
                            h_hbm.at[srcv.at[nxt]], rowb.at[b], gsem.at[b]
                        )

                return 0

            lax.fori_loop(0, hlf // nbuf, grp, 0)

        for h in range(nhalf):
            run_half(h * hlf)
        plsc.subcore_barrier()
        pltpu.sync_copy(
            acc.at[pl.ds(sid * share, share)],
            out_hbm.at[cid].at[pl.ds(sid * share, share)],
        )

    return k


def _scalar_agg_kernel(np_, tch):
    share = np_ // NS

    @functools.partial(
        pl.kernel,
        out_type=jax.ShapeDtypeStruct((NC, np_), jnp.float32),
        mesh=_mesh(),
        compiler_params=pltpu.CompilerParams(needs_layout_passes=False),
        scratch_types=[
            pltpu.VMEM((tch, CH), jnp.int32),
            pltpu.VMEM((tch, CH), jnp.int32),
            pltpu.VMEM((np_,), jnp.float32),
            pltpu.VMEM((CH,), jnp.float32),
            pltpu.VMEM((share,), jnp.float32),
            pltpu.VMEM_SHARED((np_,), jnp.float32),
        ],
    )
    def k(st_hbm, src3, dst3, out_hbm, srcv, dstv, table_v, chunk_v, obuf_v, acc):
        cid = lax.axis_index("c")
        sid = lax.axis_index("s")
        w = cid * NS + sid

        def fill_zero(i, _):
            obuf_v[pl.ds(i * 16, 16)] = jnp.zeros((16,), jnp.float32)
            return 0

        lax.fori_loop(0, share // 16, fill_zero, 0)
        pltpu.sync_copy(obuf_v, acc.at[pl.ds(sid * share, share)])
        pltpu.sync_copy(st_hbm, table_v)
        pltpu.sync_copy(src3.at[w], srcv)
        pltpu.sync_copy(dst3.at[w], dstv)
        plsc.subcore_barrier()

        def chunk(j, _):
            for kk in range(CH // 16):
                idx16 = srcv[j, pl.ds(kk * 16, 16)]
                chunk_v[pl.ds(kk * 16, 16)] = plsc.load_gather(table_v, [idx16])
            pltpu.sync_copy(chunk_v, acc.at[dstv.at[j]], add=True)
            return 0

        lax.fori_loop(0, tch, chunk, 0)
        plsc.subcore_barrier()
        pltpu.sync_copy(acc.at[pl.ds(sid * share, share)], obuf_v)
        pltpu.sync_copy(obuf_v, out_hbm.at[cid].at[pl.ds(sid * share, share)])

    return k


BK = 32768


def _edge_prep(n, e, ep, np_):
    smask = (1 << (n.bit_length() - 1)) - 1
    gmask = 127

    def body(ei_ref, src_ref, dst_ref):
        i = pl.program_id(0)
        col = jax.lax.broadcasted_iota(jnp.int32, (1, BK), 1) + i * BK
        sv = ei_ref[0:1, :]
        dv = ei_ref[1:2, :]
        p = col - e
        src = jnp.where(col < e, sv, (p * 37) & smask)
        dst = jnp.where(col < e, dv, n + (p & gmask))
        src_ref[...] = src.reshape(BK)
        dst_ref[...] = dst.reshape(BK)

    last = (e - 1) // BK
    return pl.pallas_call(
        body,
        grid=(ep // BK,),
        in_specs=[pl.BlockSpec((2, BK), lambda i: (0, jnp.minimum(i, last)))],
        out_specs=[
            pl.BlockSpec((BK,), lambda i: (i,)),
            pl.BlockSpec((BK,), lambda i: (i,)),
        ],
        out_shape=[
            jax.ShapeDtypeStruct((ep,), jnp.int32),
            jax.ShapeDtypeStruct((ep,), jnp.int32),
        ],
    )


def _scale_mm_body(x_ref, w_ref, degp_ref, h_ref, dis_ref):
    deg = 1.0 + degp_ref[0] + degp_ref[1]
    dis = lax.rsqrt(deg)
    h = jnp.dot(x_ref[...], w_ref[...], preferred_element_type=jnp.float32,
                precision=lax.Precision.HIGHEST)
    h_ref[...] = h * dis.reshape(R, 1)
    dis_ref[...] = dis


def _scale_mm(np_):
    return pl.pallas_call(
        _scale_mm_body,
        grid=(np_ // R,),
        in_specs=[
            pl.BlockSpec((R, D), lambda i: (i, 0)),
            pl.BlockSpec((D, D), lambda i: (0, 0)),
            pl.BlockSpec((NC, R), lambda i: (0, i)),
        ],
        out_specs=[
            pl.BlockSpec((R, D), lambda i: (i, 0)),
            pl.BlockSpec((R,), lambda i: (i,)),
        ],
        out_shape=[
            jax.ShapeDtypeStruct((np_, D), jnp.float32),
            jax.ShapeDtypeStruct((np_,), jnp.float32),
        ],
    )


def _mid_body(p_ref, h_ref, dis_ref, b1_ref, w2_ref, st_ref):
    a = p_ref[0] + p_ref[1] + h_ref[...]
    dis = dis_ref[...]
    h1 = dis.reshape(R, 1) * a + b1_ref[...]
    r = jnp.maximum(h1, 0.0)
    s = jnp.sum(r * w2_ref[...], axis=1)
    st_ref[...] = dis * s


def _mid(np_):
    return pl.pallas_call(
        _mid_body,
        grid=(np_ // R,),
        in_specs=[
            pl.BlockSpec((NC, R, D), lambda i: (0, i, 0)),
            pl.BlockSpec((R, D), lambda i: (i, 0)),
            pl.BlockSpec((R,), lambda i: (i,)),
            pl.BlockSpec((1, D), lambda i: (0, 0)),
            pl.BlockSpec((1, D), lambda i: (0, 0)),
        ],
        out_specs=pl.BlockSpec((R,), lambda i: (i,)),
        out_shape=jax.ShapeDtypeStruct((np_,), jnp.float32),
    )


def _final_body(p2_ref, st_ref, dis_ref, b2_ref, o_ref):
    t = p2_ref[0] + p2_ref[1] + st_ref[...]
    o = jnp.tanh(dis_ref[...] * t + b2_ref[0])
    o_ref[...] = o.reshape(R, 1)


def _final(np_, n):
    return pl.pallas_call(
        _final_body,
        grid=(np_ // R,),
        in_specs=[
            pl.BlockSpec((NC, R), lambda i: (0, i)),
            pl.BlockSpec((R,), lambda i: (i,)),
            pl.BlockSpec((R,), lambda i: (i,)),
            pl.BlockSpec((1,), lambda i: (0,)),
        ],
        out_specs=pl.BlockSpec((R, 1), lambda i: (i, 0)),
        out_shape=jax.ShapeDtypeStruct((n, 1), jnp.float32),
    )


def kernel(x, edge_index, W1, b1, W2, b2):
    n = x.shape[0]
    e = edge_index.shape[1]
    np_ = _rup(n + 128, NS * 128)
    tch = _rup(-(-e // (NW * CH)), 4)
    ep = NW * CH * tch
    npad = ep - e

    del npad
    src_f, dst_f = _edge_prep(n, e, ep, np_)(edge_index.astype(jnp.int32))
    src_p = src_f.reshape(NW, tch, CH)
    dst_p = dst_f.reshape(NW, tch, CH)

    degp = _deg_kernel(np_, tch)(dst_p)
    h_t, dis = _scale_mm(np_)(x, W1, degp)
    zrows = jnp.zeros((np_ // NS, D), jnp.float32)
    aggp = _agg_kernel(np_, tch)(h_t, zrows, src_p, dst_p)
    st = _mid(np_)(aggp, h_t, dis, b1.reshape(1, D), W2.reshape(1, D))
    agg2 = _scalar_agg_kernel(np_, tch)(st, src_p, dst_p)
    return _final(np_, n)(agg2, st, dis, b2)
